# baseline (speedup 1.0000x reference)
import functools

import jax
import jax.numpy as jnp
from jax.experimental import pallas as pl
from jax.experimental.pallas import tpu as pltpu
from jax.experimental.pallas import tpu_sc as plsc

_BLK = 2048
_NW = 32
_RING = 16


def _iota16():
    return jax.lax.iota(jnp.int32, 16)


def _sc_gather(ut_t, mt_t, user_idx, movie_idx):
    batch = user_idx.shape[0]
    emb = ut_t.shape[0]
    mesh = plsc.VectorSubcoreMesh(core_axis_name="core", subcore_axis_name="subcore")

    @functools.partial(
        pl.kernel,
        out_type=(
            jax.ShapeDtypeStruct((batch, emb), jnp.float32),
            jax.ShapeDtypeStruct((batch, emb), jnp.float32),
        ),
        mesh=mesh,
        compiler_params=pltpu.CompilerParams(
            use_tc_tiling_on_sc=True, needs_layout_passes=False),
        scratch_types=[
            pltpu.VMEM((2048,), jnp.int32),
            pltpu.VMEM((batch,), jnp.int32),
            pltpu.VMEM((272,), jnp.int32),
            pltpu.VMEM((272,), jnp.int32),
            pltpu.VMEM((272,), jnp.int32),
            pltpu.VMEM((288,), jnp.int32),
            pltpu.VMEM((5, emb, 128), jnp.float32),
            pltpu.VMEM((_RING, emb), jnp.float32),
            pltpu.SMEM((4,), jnp.int32),
            pltpu.SemaphoreType.DMA,
            pltpu.SemaphoreType.DMA,
        ],
    )
    def gather_kernel(ut_hbm, mt_hbm, ui_hbm, mi_hbm, uo_hbm, mo_hbm,
                      idx_v, hits_v, cnt_v, seg_v, ptr_v, list_v, tile_v,
                      row_v, cnt_s, sem_t, sem_r):
        wid = jax.lax.axis_index("subcore") * mesh.num_cores + jax.lax.axis_index("core")
        nvec = batch // 16

        def run_table(tab_hbm, i_hbm, o_hbm, nrows, tpw):
            lo = wid * tpw
            ntiles = (nrows + 127) // 128
            myn = jnp.maximum(0, jnp.minimum(tpw, ntiles - lo))

            ones = jnp.full((16,), 1, jnp.int32)
            zeros = jnp.full((16,), 0, jnp.int32)
            nchunk = batch // 2048

            for v in range(272 // 16):
                cnt_v[pl.ds(v * 16, 16)] = zeros

            def lanes(i):
                iv = idx_v[pl.ds(i * 16, 16)]
                ql = jax.lax.shift_right_logical(iv, 7) - lo
                m = (ql >= 0) & (ql < myn)
                qlc = jnp.clip(ql, 0, tpw - 1)
                return iv, m, qlc

            def hist_body(i, _):
                _, m, qlc = lanes(i)
                plsc.addupdate_scatter(cnt_v, [qlc], ones, mask=m)
                return 0

            for c in range(nchunk):
                pltpu.sync_copy(i_hbm.at[pl.ds(c * 2048, 2048)], idx_v)
                jax.lax.fori_loop(0, 128, hist_body, 0)

            def pfx_body(v, carry):
                hc, nzc = carry
                c16 = cnt_v[pl.ds(v * 16, 16)]
                cum = plsc.cumsum(c16)
                seg16 = hc + cum - c16
                seg_v[pl.ds(v * 16, 16)] = seg16
                ptr_v[pl.ds(v * 16, 16)] = seg16
                nzm = c16 > 0
                nzcum = plsc.cumsum(nzm.astype(jnp.int32))
                plsc.store_scatter(list_v, [nzc + nzcum - 1],
                                   v * 16 + _iota16(), mask=nzm)
                return hc + cum[15], nzc + nzcum[15]

            _, nz = jax.lax.fori_loop(0, 272 // 16, pfx_body,
                                      (jnp.int32(0), jnp.int32(0)))

            def make_place_body(c):
                def place_body(i, _):
                    iv, m, qlc = lanes(i)
                    sp = iv & 127
                    j16 = c * 2048 + i * 16 + _iota16()
                    packed = (j16 << 7) | sp
                    base = plsc.load_gather(ptr_v, [qlc])
                    occ, _2 = plsc.scan_count(qlc, mask=m)
                    pos = base + occ - 1
                    plsc.store_scatter(hits_v, [pos], packed, mask=m)
                    plsc.addupdate_scatter(ptr_v, [qlc], ones, mask=m)
                    return 0
                return place_body

            for c in range(nchunk):
                pltpu.sync_copy(i_hbm.at[pl.ds(c * 2048, 2048)], idx_v)
                jax.lax.fori_loop(0, 128, make_place_body(c), 0)

            cnt_s[0] = 0

            def start_tile(t_id, buf):
                off = pl.multiple_of((lo + t_id) * 128, 128)
                pltpu.async_copy(tab_hbm.at[:, pl.ds(off, 128)],
                                 tile_v.at[buf], sem_t)

            l0 = list_v[pl.ds(0, 16)]
            for p in range(4):
                @pl.when(nz > p)
                def _():
                    start_tile(l0[p], p)

            @pl.loop(0, tpw)
            def _(tt):
                @pl.when(tt < nz)
                def _():
                    lv = list_v[pl.ds(tt, 16)]
                    t_id = lv[0]
                    buf = tt % 5
                    pltpu.make_async_copy(
                        tab_hbm.at[:, pl.ds(0, 128)], tile_v.at[buf], sem_t
                    ).wait()

                    @pl.when(tt + 4 < nz)
                    def _():
                        start_tile(lv[4], (tt + 4) % 5)

                    bufv = jnp.full((16,), buf, jnp.int32)
                    sv = seg_v[pl.ds(t_id, 16)]
                    s0 = sv[0]
                    nh_t = sv[1] - s0

                    def hscan(h, _):
                        hv = hits_v[pl.ds(s0 + h * 16, 16)]
                        mti = ((h * 16 + _iota16()) < nh_t).astype(jnp.int32)

                        if True:

                            for k in range(16):
                                hval = hv[k]

                                @pl.when(mti[k] != 0)
                                def _():
                                    j = jax.lax.shift_right_logical(hval, 7)
                                    sp = hval & 127
                                    c = cnt_s[0]
                                    slot = c & (_RING - 1)

                                    @pl.when(c >= _RING)
                                    def _():
                                        pltpu.make_async_copy(
                                            o_hbm.at[pl.ds(0, 1)],
                                            row_v.at[pl.ds(0, 1)],
                                            sem_r).wait()

                                    spv = jnp.full((16,), sp, jnp.int32)
                                    slot_row = row_v.at[slot]
                                    for cc in range(emb // 16):
                                        rows = cc * 16 + _iota16()
                                        vals = plsc.load_gather(
                                            tile_v, [bufv, rows, spv])
                                        slot_row[pl.ds(cc * 16, 16)] = vals
                                    pltpu.async_copy(
                                        row_v.at[pl.ds(slot, 1)],
                                        o_hbm.at[pl.ds(j, 1)], sem_r)
                                    cnt_s[0] = c + 1
                        return 0

                    jax.lax.fori_loop(0, (nh_t + 15) >> 4, hscan, 0)

            rem = jnp.minimum(cnt_s[0], _RING)

            @pl.loop(0, _RING)
            def _(r):
                @pl.when(r < rem)
                def _():
                    pltpu.make_async_copy(
                        o_hbm.at[pl.ds(0, 1)], row_v.at[pl.ds(0, 1)],
                        sem_r).wait()

        def tiles_per_worker(nrows):
            return ((nrows + 127) // 128 + _NW - 1) // _NW

        run_table(ut_hbm, ui_hbm, uo_hbm, ut_t.shape[1],
                  tiles_per_worker(ut_t.shape[1]))
        run_table(mt_hbm, mi_hbm, mo_hbm, mt_t.shape[1],
                  tiles_per_worker(mt_t.shape[1]))

    return gather_kernel(ut_t, mt_t, user_idx, movie_idx)


def _mlp_body(ue, me, g, w1, b1, w2, b2, out):
    emb = ue.shape[1]
    ng = g.shape[0]
    h = jnp.dot(ue[...], w1[0:emb, :], preferred_element_type=jnp.float32)
    h += jnp.dot(me[...], w1[emb:2 * emb, :], preferred_element_type=jnp.float32)
    h += jax.lax.dot_general(g[...], w1[2 * emb:2 * emb + ng, :],
                             (((0,), (0,)), ((), ())),
                             preferred_element_type=jnp.float32)
    h = jnp.maximum(h + b1[...], 0.0)
    out[...] = jnp.dot(h, w2[...], preferred_element_type=jnp.float32) + b2[...]


def _tc_mlp(ue, me, genre_t, w1, b1, w2, b2):
    batch, emb = ue.shape
    ng = genre_t.shape[0]
    hidden = w1.shape[1]
    grid = (batch // _BLK,)
    return pl.pallas_call(
        _mlp_body,
        grid=grid,
        in_specs=[
            pl.BlockSpec((_BLK, emb), lambda i: (i, 0)),
            pl.BlockSpec((_BLK, emb), lambda i: (i, 0)),
            pl.BlockSpec((ng, _BLK), lambda i: (0, i)),
            pl.BlockSpec((2 * emb + ng, hidden), lambda i: (0, 0)),
            pl.BlockSpec((1, hidden), lambda i: (0, 0)),
            pl.BlockSpec((hidden, 1), lambda i: (0, 0)),
            pl.BlockSpec((1, 1), lambda i: (0, 0)),
        ],
        out_specs=pl.BlockSpec((_BLK, 1), lambda i: (i, 0)),
        out_shape=jax.ShapeDtypeStruct((batch, 1), jnp.float32),
    )(ue, me, genre_t, w1, b1, w2, b2)


def kernel(user, movie, genre_vec, user_table, movie_table, W1, b1, W2, b2):
    batch = user.shape[0]
    ue, me = _sc_gather(user_table.T, movie_table.T, user, movie)
    out = _tc_mlp(
        ue, me, genre_vec.T, W1,
        b1.reshape(1, -1), W2, b2.reshape(1, 1),
    )
    return out.reshape(batch)


# speedup vs baseline: 1.0211x; 1.0211x over previous
import functools

import jax
import jax.numpy as jnp
from jax.experimental import pallas as pl
from jax.experimental.pallas import tpu as pltpu
from jax.experimental.pallas import tpu_sc as plsc

_BLK = 2048
_NW = 32
_RING = 16


def _iota16():
    return jax.lax.iota(jnp.int32, 16)


def _sc_gather(ut_t, mt_t, user_idx, movie_idx):
    batch = user_idx.shape[0]
    emb = ut_t.shape[0]
    mesh = plsc.VectorSubcoreMesh(core_axis_name="core", subcore_axis_name="subcore")

    @functools.partial(
        pl.kernel,
        out_type=(
            jax.ShapeDtypeStruct((batch, emb), jnp.float32),
            jax.ShapeDtypeStruct((batch, emb), jnp.float32),
        ),
        mesh=mesh,
        compiler_params=pltpu.CompilerParams(
            use_tc_tiling_on_sc=True, needs_layout_passes=False),
        scratch_types=[
            pltpu.VMEM((batch,), jnp.int32),
            pltpu.VMEM((batch,), jnp.int32),
            pltpu.VMEM((272,), jnp.int32),
            pltpu.VMEM((272,), jnp.int32),
            pltpu.VMEM((272,), jnp.int32),
            pltpu.VMEM((288,), jnp.int32),
            pltpu.VMEM((3, emb, 128), jnp.float32),
            pltpu.VMEM((_RING, emb), jnp.float32),
            pltpu.SMEM((4,), jnp.int32),
            pltpu.SemaphoreType.DMA,
            pltpu.SemaphoreType.DMA,
        ],
    )
    def gather_kernel(ut_hbm, mt_hbm, ui_hbm, mi_hbm, uo_hbm, mo_hbm,
                      idx_v, hits_v, cnt_v, seg_v, ptr_v, list_v, tile_v,
                      row_v, cnt_s, sem_t, sem_r):
        wid = jax.lax.axis_index("subcore") * mesh.num_cores + jax.lax.axis_index("core")
        nvec = batch // 16

        def run_table(tab_hbm, i_hbm, o_hbm, nrows, tpw):
            lo = wid * tpw
            ntiles = (nrows + 127) // 128
            myn = jnp.maximum(0, jnp.minimum(tpw, ntiles - lo))

            ones = jnp.full((16,), 1, jnp.int32)
            zeros = jnp.full((16,), 0, jnp.int32)

            for v in range(272 // 16):
                cnt_v[pl.ds(v * 16, 16)] = zeros

            def lanes(i):
                iv = idx_v[pl.ds(i * 16, 16)]
                ql = jax.lax.shift_right_logical(iv, 7) - lo
                m = (ql >= 0) & (ql < myn)
                qlc = jnp.clip(ql, 0, tpw - 1)
                return iv, m, qlc

            def hist_body(i, _):
                _, m, qlc = lanes(i)
                plsc.addupdate_scatter(cnt_v, [qlc], ones, mask=m)
                return 0

            pltpu.sync_copy(i_hbm, idx_v)
            jax.lax.fori_loop(0, nvec, hist_body, 0)

            def pfx_body(v, carry):
                hc, nzc = carry
                c16 = cnt_v[pl.ds(v * 16, 16)]
                cum = plsc.cumsum(c16)
                seg16 = hc + cum - c16
                seg_v[pl.ds(v * 16, 16)] = seg16
                ptr_v[pl.ds(v * 16, 16)] = seg16
                nzm = c16 > 0
                nzcum = plsc.cumsum(nzm.astype(jnp.int32))
                plsc.store_scatter(list_v, [nzc + nzcum - 1],
                                   v * 16 + _iota16(), mask=nzm)
                return hc + cum[15], nzc + nzcum[15]

            _, nz = jax.lax.fori_loop(0, 272 // 16, pfx_body,
                                      (jnp.int32(0), jnp.int32(0)))

            def place_body(i, _):
                iv, m, qlc = lanes(i)
                sp = iv & 127
                j16 = i * 16 + _iota16()
                packed = (j16 << 7) | sp
                base = plsc.load_gather(ptr_v, [qlc])
                occ, _2 = plsc.scan_count(qlc, mask=m)
                pos = base + occ - 1
                plsc.store_scatter(hits_v, [pos], packed, mask=m)
                plsc.addupdate_scatter(ptr_v, [qlc], ones, mask=m)
                return 0

            jax.lax.fori_loop(0, nvec, place_body, 0)

            cnt_s[0] = 0

            def start_tile(t_id, buf):
                off = pl.multiple_of((lo + t_id) * 128, 128)
                pltpu.async_copy(tab_hbm.at[:, pl.ds(off, 128)],
                                 tile_v.at[buf], sem_t)

            l0 = list_v[pl.ds(0, 16)]
            for p in range(2):
                @pl.when(nz > p)
                def _():
                    start_tile(l0[p], p)

            @pl.loop(0, tpw)
            def _(tt):
                @pl.when(tt < nz)
                def _():
                    lv = list_v[pl.ds(tt, 16)]
                    t_id = lv[0]
                    buf = tt % 3
                    pltpu.make_async_copy(
                        tab_hbm.at[:, pl.ds(0, 128)], tile_v.at[buf], sem_t
                    ).wait()

                    @pl.when(tt + 2 < nz)
                    def _():
                        start_tile(lv[2], (tt + 2) % 3)

                    bufv = jnp.full((16,), buf, jnp.int32)
                    sv = seg_v[pl.ds(t_id, 16)]
                    s0 = sv[0]
                    nh_t = sv[1] - s0

                    def hscan(h, _):
                        hv = hits_v[pl.ds(s0 + h * 16, 16)]
                        mti = ((h * 16 + _iota16()) < nh_t).astype(jnp.int32)

                        if True:

                            for k in range(16):
                                hval = hv[k]

                                @pl.when(mti[k] != 0)
                                def _():
                                    j = jax.lax.shift_right_logical(hval, 7)
                                    sp = hval & 127
                                    c = cnt_s[0]
                                    slot = c & (_RING - 1)

                                    @pl.when(c >= _RING)
                                    def _():
                                        pltpu.make_async_copy(
                                            o_hbm.at[pl.ds(0, 1)],
                                            row_v.at[pl.ds(0, 1)],
                                            sem_r).wait()

                                    spv = jnp.full((16,), sp, jnp.int32)
                                    slot_row = row_v.at[slot]
                                    for cc in range(emb // 16):
                                        rows = cc * 16 + _iota16()
                                        vals = plsc.load_gather(
                                            tile_v, [bufv, rows, spv])
                                        slot_row[pl.ds(cc * 16, 16)] = vals
                                    pltpu.async_copy(
                                        row_v.at[pl.ds(slot, 1)],
                                        o_hbm.at[pl.ds(j, 1)], sem_r)
                                    cnt_s[0] = c + 1
                        return 0

                    jax.lax.fori_loop(0, (nh_t + 15) >> 4, hscan, 0)

            rem = jnp.minimum(cnt_s[0], _RING)

            @pl.loop(0, _RING)
            def _(r):
                @pl.when(r < rem)
                def _():
                    pltpu.make_async_copy(
                        o_hbm.at[pl.ds(0, 1)], row_v.at[pl.ds(0, 1)],
                        sem_r).wait()

        def tiles_per_worker(nrows):
            return ((nrows + 127) // 128 + _NW - 1) // _NW

        run_table(ut_hbm, ui_hbm, uo_hbm, ut_t.shape[1],
                  tiles_per_worker(ut_t.shape[1]))
        run_table(mt_hbm, mi_hbm, mo_hbm, mt_t.shape[1],
                  tiles_per_worker(mt_t.shape[1]))

    return gather_kernel(ut_t, mt_t, user_idx, movie_idx)


def _mlp_body(ue, me, g, w1, b1, w2, b2, out):
    emb = ue.shape[1]
    ng = g.shape[0]
    h = jnp.dot(ue[...], w1[0:emb, :], preferred_element_type=jnp.float32)
    h += jnp.dot(me[...], w1[emb:2 * emb, :], preferred_element_type=jnp.float32)
    h += jax.lax.dot_general(g[...], w1[2 * emb:2 * emb + ng, :],
                             (((0,), (0,)), ((), ())),
                             preferred_element_type=jnp.float32)
    h = jnp.maximum(h + b1[...], 0.0)
    out[...] = jnp.dot(h, w2[...], preferred_element_type=jnp.float32) + b2[...]


def _tc_mlp(ue, me, genre_t, w1, b1, w2, b2):
    batch, emb = ue.shape
    ng = genre_t.shape[0]
    hidden = w1.shape[1]
    grid = (batch // _BLK,)
    return pl.pallas_call(
        _mlp_body,
        grid=grid,
        in_specs=[
            pl.BlockSpec((_BLK, emb), lambda i: (i, 0)),
            pl.BlockSpec((_BLK, emb), lambda i: (i, 0)),
            pl.BlockSpec((ng, _BLK), lambda i: (0, i)),
            pl.BlockSpec((2 * emb + ng, hidden), lambda i: (0, 0)),
            pl.BlockSpec((1, hidden), lambda i: (0, 0)),
            pl.BlockSpec((hidden, 1), lambda i: (0, 0)),
            pl.BlockSpec((1, 1), lambda i: (0, 0)),
        ],
        out_specs=pl.BlockSpec((_BLK, 1), lambda i: (i, 0)),
        out_shape=jax.ShapeDtypeStruct((batch, 1), jnp.float32),
    )(ue, me, genre_t, w1, b1, w2, b2)


def kernel(user, movie, genre_vec, user_table, movie_table, W1, b1, W2, b2):
    batch = user.shape[0]
    ue, me = _sc_gather(user_table.T, movie_table.T, user, movie)
    out = _tc_mlp(
        ue, me, genre_vec.T, W1,
        b1.reshape(1, -1), W2, b2.reshape(1, 1),
    )
    return out.reshape(batch)
